# Initial kernel scaffold; baseline (speedup 1.0000x reference)
import jax
import jax.numpy as jnp
from jax.experimental import pallas as pl


def kernel(inputs):
    raise NotImplementedError("write your pallas kernel here")


# speedup vs baseline: 2.5743x; 2.5743x over previous
import functools

import jax
import jax.numpy as jnp
from jax import lax
from jax.experimental import pallas as pl
from jax.experimental.pallas import tpu as pltpu
from jax.experimental.pallas import tpu_sc as plsc

_BOUNDS = (-2.0, -1.5, -1.0, -0.5, 0.0, 0.5, 1.0, 1.5, 2.0)

_ROWS, _COLS = 16384, 26
_TOTAL = _ROWS * _COLS
_NC, _NS, _L = 2, 16, 16
_NW = _NC * _NS
_PER_W = _TOTAL // _NW
_VECS = _PER_W // _L
_UNROLL = 8

_mesh = plsc.VectorSubcoreMesh(core_axis_name="c", subcore_axis_name="s")


@functools.partial(
    pl.kernel,
    mesh=_mesh,
    out_type=jax.ShapeDtypeStruct((_TOTAL,), jnp.int32),
    scratch_types=[
        pltpu.VMEM((_PER_W,), jnp.float32),
        pltpu.VMEM((_PER_W,), jnp.int32),
    ],
)
def _bucketize_sc(x_hbm, out_hbm, x_v, o_v):
    wid = lax.axis_index("s") * _NC + lax.axis_index("c")
    base = wid * _PER_W
    pltpu.sync_copy(x_hbm.at[pl.ds(base, _PER_W)], x_v)

    bvecs = [jnp.full((_L,), b, jnp.float32) for b in _BOUNDS]
    one = jnp.ones((_L,), jnp.int32)
    zero = jnp.zeros((_L,), jnp.int32)

    def body(i, carry):
        b0 = i * (_L * _UNROLL)
        for u in range(_UNROLL):
            x = x_v[pl.ds(b0 + u * _L, _L)]
            acc = zero
            for bv in bvecs:
                acc = acc + jnp.where(x > bv, one, zero)
            o_v[pl.ds(b0 + u * _L, _L)] = acc
        return carry

    lax.fori_loop(0, _VECS // _UNROLL, body, 0)
    pltpu.sync_copy(o_v, out_hbm.at[pl.ds(base, _PER_W)])


def kernel(inputs):
    flat = inputs.reshape(_TOTAL)
    out = _bucketize_sc(flat)
    return out.reshape(_ROWS, _COLS)


# speedup vs baseline: 3.3786x; 1.3124x over previous
import functools

import jax
import jax.numpy as jnp
from jax import lax
from jax.experimental import pallas as pl
from jax.experimental.pallas import tpu as pltpu
from jax.experimental.pallas import tpu_sc as plsc

_BOUNDS = (-2.0, -1.5, -1.0, -0.5, 0.0, 0.5, 1.0, 1.5, 2.0)

_ROWS, _COLS = 16384, 26
_NC, _NS, _L = 2, 16, 16
_NW = _NC * _NS
_ROWS_W = _ROWS // _NW
_CHUNK = 128
_UNROLL = 1

_mesh = plsc.VectorSubcoreMesh(core_axis_name="c", subcore_axis_name="s")


@functools.partial(
    pl.kernel,
    mesh=_mesh,
    out_type=jax.ShapeDtypeStruct((_ROWS, _COLS), jnp.int32),
    scratch_types=[
        pltpu.VMEM((_CHUNK, _COLS), jnp.float32),
        pltpu.VMEM((_CHUNK, _COLS), jnp.int32),
    ],
)
def _bucketize_sc(x_hbm, out_hbm, x_v, o_v):
    wid = lax.axis_index("s") * _NC + lax.axis_index("c")
    r0 = wid * _ROWS_W

    bvecs = [jnp.full((_L,), b, jnp.float32) for b in _BOUNDS]
    one = jnp.ones((_L,), jnp.int32)
    zero = jnp.zeros((_L,), jnp.int32)

    def bucketize(x):
        acc = zero
        for bv in bvecs:
            acc = acc + jnp.where(x > bv, one, zero)
        return acc

    def chunk(c, carry):
        base = r0 + c * _CHUNK
        pltpu.sync_copy(x_hbm.at[pl.ds(base, _CHUNK)], x_v)

        def body(i, carry2):
            for u in range(_UNROLL):
                r = i * _UNROLL + u
                o_v[r, pl.ds(0, _L)] = bucketize(x_v[r, pl.ds(0, _L)])
                o_v[r, pl.ds(_COLS - _L, _L)] = bucketize(
                    x_v[r, pl.ds(_COLS - _L, _L)])
            return carry2

        lax.fori_loop(0, _CHUNK // _UNROLL, body, 0)
        pltpu.sync_copy(o_v, out_hbm.at[pl.ds(base, _CHUNK)])
        return carry

    lax.fori_loop(0, _ROWS_W // _CHUNK, chunk, 0)


def kernel(inputs):
    return _bucketize_sc(inputs)
